# baseline (speedup 1.0000x reference)
import functools

import jax
import jax.numpy as jnp
from jax import lax
from jax.experimental import pallas as pl
from jax.experimental.pallas import tpu as pltpu
from jax.experimental.pallas import tpu_sc as plsc

N = 10000
F = 128
E = 320000

NC = 2
NS = 16
L = 16
NW = NC * NS
EPW = E // NW
B = 128
NCH = 80
EPWP = NCH * B
NP = 10240
NMETA = 4


def _sc_body(x_hbm, meta_hbm, w_hbm, agg_out, deg_out,
             r0, r1, m0, m1, m2, m3, w0, w1, w2, w3,
             acc_s, deg_s, zidx,
             g0, g1, s0, s1, d0, d1,
             gm0, gm1, gm2, gm3, gw0, gw1, gw2, gw3):
    cid = lax.axis_index("c")
    sid = lax.axis_index("s")
    wid = sid * NC + cid
    rows = [r0, r1]
    metab = [m0, m1, m2, m3]
    wbuf = [w0, w1, w2, w3]
    gs = [g0, g1]
    ss = [s0, s1]
    dsem = [d0, d1]
    gm = [gm0, gm1, gm2, gm3]
    gw = [gw0, gw1, gw2, gw3]

    zero16 = jnp.zeros((L,), jnp.float32)

    def _zrow(i, carry):
        for j in range(F // L):
            r0[i, pl.ds(j * L, L)] = zero16
        return carry
    lax.fori_loop(0, B, _zrow, 0)
    zero16i = jnp.zeros((L,), jnp.int32)
    for j in range(B // L):
        zidx[pl.ds(j * L, L)] = zero16i
    for t in range(NP // B // NS):
        zc = sid * (NP // B // NS) + t
        pltpu.sync_copy(r0, acc_s.at[pl.ds(zc * B, B)])
        pltpu.sync_copy(r0.at[0], deg_s.at[pl.ds(zc * B, B)])

    plsc.subcore_barrier()

    def _meta(c, i):
        pltpu.async_copy(meta_hbm.at[wid, c], metab[i], gm[i])
        pltpu.async_copy(w_hbm.at[wid, c], wbuf[i], gw[i])

    def _wait_meta(c, i):
        pltpu.make_async_copy(meta_hbm.at[wid, c], metab[i], gm[i]).wait()
        pltpu.make_async_copy(w_hbm.at[wid, c], wbuf[i], gw[i]).wait()

    def _gather(mi, buf, sem):
        return pltpu.async_copy(x_hbm.at[zidx], buf, sem)

    def _wait_gather(mi, buf, sem):
        pltpu.make_async_copy(x_hbm.at[zidx], buf, sem).wait()

    _meta(0, 0)
    _meta(1, 1)
    _wait_meta(0, 0)
    _gather(0, r0, g0)

    def _chunk_iter(k, carry):
        for i in range(NMETA):
            c = k * NMETA + i
            p = i % 2
            np_ = (i + 1) % 2
            mi = i
            mn = (i + 1) % NMETA
            m2_ = (i + 2) % NMETA

            _wait_gather(mi, rows[p], gs[p])

            @pl.when(c + 2 < NCH)
            def _():
                _meta(c + 2, m2_)

            @pl.when(c + 1 < NCH)
            def _():
                @pl.when(c >= 1)
                def _():
                    pltpu.make_async_copy(
                        rows[np_], acc_s.at[metab[mn].at[1]], ss[np_]).wait()
                    pltpu.make_async_copy(
                        wbuf[mn], deg_s.at[metab[mn].at[1]],
                        dsem[np_]).wait()
                _wait_meta(c + 1, mn)
                _gather(mn, rows[np_], gs[np_])

            def _grp(g, carry2):
                for k2 in range(L):
                    e = g * L + k2
                    wb = plsc.load_gather(
                        wbuf[mi], [jnp.full((L,), e, jnp.int32)])
                    for j in range(F // L):
                        rows[p][e, pl.ds(j * L, L)] = (
                            rows[p][e, pl.ds(j * L, L)] * wb)
                return carry2
            lax.fori_loop(0, B // L, _grp, 0)

            pltpu.async_copy(
                wbuf[mi], deg_s.at[metab[mi].at[1]], dsem[p], add=True)
            pltpu.async_copy(
                rows[p], acc_s.at[metab[mi].at[1]], ss[p], add=True)
        return carry
    lax.fori_loop(0, NCH // NMETA, _chunk_iter, 0)

    for p in range(2):
        pltpu.make_async_copy(
            rows[p], acc_s.at[metab[0].at[1]], ss[p]).wait()
        pltpu.make_async_copy(
            wbuf[0], deg_s.at[metab[0].at[1]], dsem[p]).wait()

    plsc.subcore_barrier()

    @pl.when(sid == 0)
    def _():
        pltpu.sync_copy(acc_s.at[pl.ds(0, N)], agg_out.at[cid])
        pltpu.sync_copy(deg_s, deg_out.at[cid])


_sc_agg = functools.partial(
    pl.kernel,
    out_type=(
        jax.ShapeDtypeStruct((NC, N, F), jnp.float32),
        jax.ShapeDtypeStruct((NC, NP), jnp.float32),
    ),
    mesh=plsc.VectorSubcoreMesh(core_axis_name="c", subcore_axis_name="s"),
    compiler_params=pltpu.CompilerParams(needs_layout_passes=False),
    scratch_types=(
        [pltpu.VMEM((B, F), jnp.float32)] * 2
        + [pltpu.VMEM((2, B), jnp.int32)] * NMETA
        + [pltpu.VMEM((B,), jnp.float32)] * NMETA
        + [
            pltpu.VMEM_SHARED((NP, F), jnp.float32),
            pltpu.VMEM_SHARED((NP,), jnp.float32),
            pltpu.VMEM((B,), jnp.int32),
        ]
        + [pltpu.SemaphoreType.DMA] * (6 + 2 * NMETA)
    ),
)(_sc_body)


RB = 1000


def _tc_body(x_ref, a0_ref, a1_ref, d0_ref, d1_ref,
             az_ref, bz_ref, ah_ref, bh_ref, vz_ref, vh_ref, o_ref):
    x = x_ref[...]
    agg = a0_ref[...] + a1_ref[...]
    deg = d0_ref[...] + d1_ref[...]
    deg_inv = jnp.where(deg > 0, 1.0 / deg, 0.0)
    agg = agg * deg_inv
    pz = (jnp.dot(x, az_ref[...], preferred_element_type=jnp.float32)
          + jnp.dot(agg, bz_ref[...], preferred_element_type=jnp.float32)
          + vz_ref[...])
    ph = (jnp.dot(x, ah_ref[...], preferred_element_type=jnp.float32)
          + jnp.dot(agg, bh_ref[...], preferred_element_type=jnp.float32)
          + vh_ref[...])
    z = jax.nn.sigmoid(pz)
    ht = jnp.tanh(ph)
    o_ref[...] = jnp.maximum((1.0 - z) * ht, 0.0)


def _tc_gru(x, a0, a1, d0, d1, az, bz, ah, bh, vz, vh):
    grid = (N // RB,)
    row = pl.BlockSpec((RB, F), lambda i: (i, 0))
    col = pl.BlockSpec((RB, 1), lambda i: (i, 0))
    full = pl.BlockSpec((F, F), lambda i: (0, 0))
    vec = pl.BlockSpec((1, F), lambda i: (0, 0))
    return pl.pallas_call(
        _tc_body,
        grid=grid,
        in_specs=[row, row, row, col, col, full, full, full, full, vec, vec],
        out_specs=row,
        out_shape=jax.ShapeDtypeStruct((N, F), jnp.float32),
    )(x, a0, a1, d0, d1, az, bz, ah, bh, vz, vh)


def kernel(x, edge_index, edge_weight,
           W0_z, W1_z, b_z, W0_r, W1_r, b_r, W0_h, W1_h, b_h):
    pad = EPWP - EPW
    src = jnp.pad(edge_index[0].reshape(NW, EPW), ((0, 0), (0, pad)))
    dst = jnp.pad(edge_index[1].reshape(NW, EPW), ((0, 0), (0, pad)))
    w = jnp.pad(edge_weight.reshape(NW, EPW), ((0, 0), (0, pad)))
    meta = jnp.stack(
        [src.reshape(NW, NCH, B), dst.reshape(NW, NCH, B)], axis=2)
    agg_parts, deg_parts = _sc_agg(x, meta, w.reshape(NW, NCH, B))
    return _tc_gru(
        x, agg_parts[0], agg_parts[1],
        deg_parts[0][:N, None], deg_parts[1][:N, None],
        W0_z[:F], W1_z[:F], W0_h[:F], W1_h[:F],
        b_z[None, :], b_h[None, :])


# speedup vs baseline: 23.8690x; 23.8690x over previous
import functools

import jax
import jax.numpy as jnp
from jax import lax
from jax.experimental import pallas as pl
from jax.experimental.pallas import tpu as pltpu
from jax.experimental.pallas import tpu_sc as plsc

N = 10000
F = 128
E = 320000

NC = 2
NS = 16
L = 16
NW = NC * NS
EPW = E // NW
B = 128
NCH = 80
EPWP = NCH * B
NP = 10240
NMETA = 4


def _sc_body(x_hbm, meta_hbm, w_hbm, agg_out, deg_out,
             r0, r1, m0, m1, m2, m3, w0, w1, w2, w3,
             acc_s, deg_s,
             g0, g1, s0, s1, d0, d1, g0b, g1b,
             gm0, gm1, gm2, gm3, gw0, gw1, gw2, gw3):
    cid = lax.axis_index("c")
    sid = lax.axis_index("s")
    wid = sid * NC + cid
    rows = [r0, r1]
    metab = [m0, m1, m2, m3]
    wbuf = [w0, w1, w2, w3]
    gs = [(g0, g0b), (g1, g1b)]
    ss = [s0, s1]
    dsem = [d0, d1]
    gm = [gm0, gm1, gm2, gm3]
    gw = [gw0, gw1, gw2, gw3]

    zero16 = jnp.zeros((L,), jnp.float32)

    def _zrow(i, carry):
        for j in range(F // L):
            r0[i, pl.ds(j * L, L)] = zero16
        return carry
    lax.fori_loop(0, B, _zrow, 0)
    for t in range(NP // B // NS):
        zc = sid * (NP // B // NS) + t
        pltpu.sync_copy(r0, acc_s.at[pl.ds(zc * B, B)])
        pltpu.sync_copy(r0.at[0], deg_s.at[pl.ds(zc * B, B)])

    plsc.subcore_barrier()

    def _meta(c, i):
        pltpu.async_copy(meta_hbm.at[wid, c], metab[i], gm[i])
        pltpu.async_copy(w_hbm.at[wid, c], wbuf[i], gw[i])

    def _wait_meta(c, i):
        pltpu.make_async_copy(meta_hbm.at[wid, c], metab[i], gm[i]).wait()
        pltpu.make_async_copy(w_hbm.at[wid, c], wbuf[i], gw[i]).wait()

    HB = B // 2

    def _gather(mi, buf, sem):
        pltpu.async_copy(
            x_hbm.at[metab[mi].at[0, pl.ds(0, HB)]],
            buf.at[pl.ds(0, HB)], sem[0])
        pltpu.async_copy(
            x_hbm.at[metab[mi].at[0, pl.ds(HB, HB)]],
            buf.at[pl.ds(HB, HB)], sem[1])

    def _wait_gather(mi, buf, sem):
        pltpu.make_async_copy(
            x_hbm.at[metab[mi].at[0, pl.ds(0, HB)]],
            buf.at[pl.ds(0, HB)], sem[0]).wait()
        pltpu.make_async_copy(
            x_hbm.at[metab[mi].at[0, pl.ds(HB, HB)]],
            buf.at[pl.ds(HB, HB)], sem[1]).wait()

    _meta(0, 0)
    _meta(1, 1)
    _wait_meta(0, 0)
    _gather(0, r0, gs[0])

    def _chunk_iter(k, carry):
        for i in range(NMETA):
            c = k * NMETA + i
            p = i % 2
            np_ = (i + 1) % 2
            mi = i
            mn = (i + 1) % NMETA
            m2_ = (i + 2) % NMETA

            _wait_gather(mi, rows[p], gs[p])

            @pl.when(c + 2 < NCH)
            def _():
                _meta(c + 2, m2_)

            @pl.when(c + 1 < NCH)
            def _():
                @pl.when(c >= 1)
                def _():
                    pltpu.make_async_copy(
                        rows[np_], acc_s.at[metab[mn].at[1]], ss[np_]).wait()
                    pltpu.make_async_copy(
                        wbuf[mn], deg_s.at[metab[mn].at[1]],
                        dsem[np_]).wait()
                _wait_meta(c + 1, mn)
                _gather(mn, rows[np_], gs[np_])

            def _grp(g, carry2):
                for k2 in range(L):
                    e = g * L + k2
                    wb = plsc.load_gather(
                        wbuf[mi], [jnp.full((L,), e, jnp.int32)])
                    for j in range(F // L):
                        rows[p][e, pl.ds(j * L, L)] = (
                            rows[p][e, pl.ds(j * L, L)] * wb)
                return carry2
            lax.fori_loop(0, B // L, _grp, 0)

            pltpu.async_copy(
                wbuf[mi], deg_s.at[metab[mi].at[1]], dsem[p], add=True)
            pltpu.async_copy(
                rows[p], acc_s.at[metab[mi].at[1]], ss[p], add=True)
        return carry
    lax.fori_loop(0, NCH // NMETA, _chunk_iter, 0)

    for p in range(2):
        pltpu.make_async_copy(
            rows[p], acc_s.at[metab[0].at[1]], ss[p]).wait()
        pltpu.make_async_copy(
            wbuf[0], deg_s.at[metab[0].at[1]], dsem[p]).wait()

    plsc.subcore_barrier()

    @pl.when(sid == 0)
    def _():
        pltpu.sync_copy(acc_s.at[pl.ds(0, N)], agg_out.at[cid])
        pltpu.sync_copy(deg_s, deg_out.at[cid])


_sc_agg = functools.partial(
    pl.kernel,
    out_type=(
        jax.ShapeDtypeStruct((NC, N, F), jnp.float32),
        jax.ShapeDtypeStruct((NC, NP), jnp.float32),
    ),
    mesh=plsc.VectorSubcoreMesh(core_axis_name="c", subcore_axis_name="s"),
    compiler_params=pltpu.CompilerParams(needs_layout_passes=False),
    scratch_types=(
        [pltpu.VMEM((B, F), jnp.float32)] * 2
        + [pltpu.VMEM((2, B), jnp.int32)] * NMETA
        + [pltpu.VMEM((B,), jnp.float32)] * NMETA
        + [
            pltpu.VMEM_SHARED((NP, F), jnp.float32),
            pltpu.VMEM_SHARED((NP,), jnp.float32),
        ]
        + [pltpu.SemaphoreType.DMA] * (8 + 2 * NMETA)
    ),
)(_sc_body)


RB = 1000


def _tc_body(x_ref, a0_ref, a1_ref, d0_ref, d1_ref,
             az_ref, bz_ref, ah_ref, bh_ref, vz_ref, vh_ref, o_ref):
    x = x_ref[...]
    agg = a0_ref[...] + a1_ref[...]
    deg = d0_ref[...] + d1_ref[...]
    deg_inv = jnp.where(deg > 0, 1.0 / deg, 0.0)
    agg = agg * deg_inv
    pz = (jnp.dot(x, az_ref[...], preferred_element_type=jnp.float32)
          + jnp.dot(agg, bz_ref[...], preferred_element_type=jnp.float32)
          + vz_ref[...])
    ph = (jnp.dot(x, ah_ref[...], preferred_element_type=jnp.float32)
          + jnp.dot(agg, bh_ref[...], preferred_element_type=jnp.float32)
          + vh_ref[...])
    z = jax.nn.sigmoid(pz)
    ht = jnp.tanh(ph)
    o_ref[...] = jnp.maximum((1.0 - z) * ht, 0.0)


def _tc_gru(x, a0, a1, d0, d1, az, bz, ah, bh, vz, vh):
    grid = (N // RB,)
    row = pl.BlockSpec((RB, F), lambda i: (i, 0))
    col = pl.BlockSpec((RB, 1), lambda i: (i, 0))
    full = pl.BlockSpec((F, F), lambda i: (0, 0))
    vec = pl.BlockSpec((1, F), lambda i: (0, 0))
    return pl.pallas_call(
        _tc_body,
        grid=grid,
        in_specs=[row, row, row, col, col, full, full, full, full, vec, vec],
        out_specs=row,
        out_shape=jax.ShapeDtypeStruct((N, F), jnp.float32),
    )(x, a0, a1, d0, d1, az, bz, ah, bh, vz, vh)


def kernel(x, edge_index, edge_weight,
           W0_z, W1_z, b_z, W0_r, W1_r, b_r, W0_h, W1_h, b_h):
    pad = EPWP - EPW
    src = jnp.pad(edge_index[0].reshape(NW, EPW), ((0, 0), (0, pad)))
    dst = jnp.pad(edge_index[1].reshape(NW, EPW), ((0, 0), (0, pad)))
    w = jnp.pad(edge_weight.reshape(NW, EPW), ((0, 0), (0, pad)))
    meta = jnp.stack(
        [src.reshape(NW, NCH, B), dst.reshape(NW, NCH, B)], axis=2)
    agg_parts, deg_parts = _sc_agg(x, meta, w.reshape(NW, NCH, B))
    return _tc_gru(
        x, agg_parts[0], agg_parts[1],
        deg_parts[0][:N, None], deg_parts[1][:N, None],
        W0_z[:F], W1_z[:F], W0_h[:F], W1_h[:F],
        b_z[None, :], b_h[None, :])


# speedup vs baseline: 31.2717x; 1.3101x over previous
import functools

import jax
import jax.numpy as jnp
from jax import lax
from jax.experimental import pallas as pl
from jax.experimental.pallas import tpu as pltpu
from jax.experimental.pallas import tpu_sc as plsc

N = 10000
F = 128
E = 320000

NC = 2
NS = 16
L = 16
NW = NC * NS
EPW = E // NW
B = 128
NCH = 80
EPWP = NCH * B
NP = 10240
NMETA = 4


def _sc_body(x_hbm, meta_hbm, w_hbm, agg_out, deg_out,
             r0, r1, m0, m1, m2, m3, w0, w1, w2, w3,
             acc_s, deg_s,
             g0, g1, s0, s1, d0, d1,
             gm0, gm1, gm2, gm3, gw0, gw1, gw2, gw3):
    cid = lax.axis_index("c")
    sid = lax.axis_index("s")
    wid = sid * NC + cid
    rows = [r0, r1]
    metab = [m0, m1, m2, m3]
    wbuf = [w0, w1, w2, w3]
    gs = [g0, g1]
    ss = [s0, s1]
    dsem = [d0, d1]
    gm = [gm0, gm1, gm2, gm3]
    gw = [gw0, gw1, gw2, gw3]

    zero16 = jnp.zeros((L,), jnp.float32)

    def _zrow(i, carry):
        for j in range(F // L):
            r0[i, pl.ds(j * L, L)] = zero16
        return carry
    lax.fori_loop(0, B, _zrow, 0)
    for t in range(NP // B // NS):
        zc = sid * (NP // B // NS) + t
        pltpu.sync_copy(r0, acc_s.at[pl.ds(zc * B, B)])
        pltpu.sync_copy(r0.at[0], deg_s.at[pl.ds(zc * B, B)])

    plsc.subcore_barrier()

    def _meta(c, i):
        pltpu.async_copy(meta_hbm.at[wid, c], metab[i], gm[i])
        pltpu.async_copy(w_hbm.at[wid, c], wbuf[i], gw[i])

    def _wait_meta(c, i):
        pltpu.make_async_copy(meta_hbm.at[wid, c], metab[i], gm[i]).wait()
        pltpu.make_async_copy(w_hbm.at[wid, c], wbuf[i], gw[i]).wait()

    def _gather(mi, buf, sem):
        return pltpu.async_copy(x_hbm.at[metab[mi].at[0]], buf, sem)

    def _wait_gather(mi, buf, sem):
        pltpu.make_async_copy(x_hbm.at[metab[mi].at[0]], buf, sem).wait()

    _meta(0, 0)
    _meta(1, 1)
    _wait_meta(0, 0)
    _gather(0, r0, g0)

    def _chunk_iter(k, carry):
        for i in range(NMETA):
            c = k * NMETA + i
            p = i % 2
            np_ = (i + 1) % 2
            mi = i
            mn = (i + 1) % NMETA
            m2_ = (i + 2) % NMETA

            _wait_gather(mi, rows[p], gs[p])

            @pl.when(c + 2 < NCH)
            def _():
                _meta(c + 2, m2_)

            @pl.when(c + 1 < NCH)
            def _():
                @pl.when(c >= 1)
                def _():
                    pltpu.make_async_copy(
                        rows[np_], acc_s.at[metab[mn].at[1]], ss[np_]).wait()
                    pltpu.make_async_copy(
                        wbuf[mn], deg_s.at[metab[mn].at[1]],
                        dsem[np_]).wait()
                _wait_meta(c + 1, mn)
                _gather(mn, rows[np_], gs[np_])

            def _grp(g, carry2):
                for k2 in range(L):
                    e = g * L + k2
                    wb = plsc.load_gather(
                        wbuf[mi], [jnp.full((L,), e, jnp.int32)])
                    for j in range(F // L):
                        rows[p][e, pl.ds(j * L, L)] = (
                            rows[p][e, pl.ds(j * L, L)] * wb)
                return carry2
            lax.fori_loop(0, B // L, _grp, 0)

            pltpu.async_copy(
                wbuf[mi], deg_s.at[metab[mi].at[1]], dsem[p], add=True)
            pltpu.async_copy(
                rows[p], acc_s.at[metab[mi].at[1]], ss[p], add=True)
        return carry
    lax.fori_loop(0, NCH // NMETA, _chunk_iter, 0)

    for p in range(2):
        pltpu.make_async_copy(
            rows[p], acc_s.at[metab[0].at[1]], ss[p]).wait()
        pltpu.make_async_copy(
            wbuf[0], deg_s.at[metab[0].at[1]], dsem[p]).wait()

    plsc.subcore_barrier()

    @pl.when(sid == 0)
    def _():
        pltpu.sync_copy(acc_s.at[pl.ds(0, N)], agg_out.at[cid])
        pltpu.sync_copy(deg_s, deg_out.at[cid])


_sc_agg = functools.partial(
    pl.kernel,
    out_type=(
        jax.ShapeDtypeStruct((NC, N, F), jnp.float32),
        jax.ShapeDtypeStruct((NC, NP), jnp.float32),
    ),
    mesh=plsc.VectorSubcoreMesh(core_axis_name="c", subcore_axis_name="s"),
    compiler_params=pltpu.CompilerParams(needs_layout_passes=False),
    scratch_types=(
        [pltpu.VMEM((B, F), jnp.float32)] * 2
        + [pltpu.VMEM((2, B), jnp.int32)] * NMETA
        + [pltpu.VMEM((B,), jnp.float32)] * NMETA
        + [
            pltpu.VMEM_SHARED((NP, F), jnp.float32),
            pltpu.VMEM_SHARED((NP,), jnp.float32),
        ]
        + [pltpu.SemaphoreType.DMA] * (6 + 2 * NMETA)
    ),
)(_sc_body)


RB = 1000


def _tc_body(x_ref, a0_ref, a1_ref, d0_ref, d1_ref,
             az_ref, bz_ref, ah_ref, bh_ref, vz_ref, vh_ref, o_ref):
    x = x_ref[...]
    agg = a0_ref[...] + a1_ref[...]
    deg = d0_ref[...] + d1_ref[...]
    deg_inv = jnp.where(deg > 0, 1.0 / deg, 0.0)
    agg = agg * deg_inv
    pz = (jnp.dot(x, az_ref[...], preferred_element_type=jnp.float32)
          + jnp.dot(agg, bz_ref[...], preferred_element_type=jnp.float32)
          + vz_ref[...])
    ph = (jnp.dot(x, ah_ref[...], preferred_element_type=jnp.float32)
          + jnp.dot(agg, bh_ref[...], preferred_element_type=jnp.float32)
          + vh_ref[...])
    z = jax.nn.sigmoid(pz)
    ht = jnp.tanh(ph)
    o_ref[...] = jnp.maximum((1.0 - z) * ht, 0.0)


def _tc_gru(x, a0, a1, d0, d1, az, bz, ah, bh, vz, vh):
    grid = (N // RB,)
    row = pl.BlockSpec((RB, F), lambda i: (i, 0))
    col = pl.BlockSpec((RB, 1), lambda i: (i, 0))
    full = pl.BlockSpec((F, F), lambda i: (0, 0))
    vec = pl.BlockSpec((1, F), lambda i: (0, 0))
    return pl.pallas_call(
        _tc_body,
        grid=grid,
        in_specs=[row, row, row, col, col, full, full, full, full, vec, vec],
        out_specs=row,
        out_shape=jax.ShapeDtypeStruct((N, F), jnp.float32),
    )(x, a0, a1, d0, d1, az, bz, ah, bh, vz, vh)


def kernel(x, edge_index, edge_weight,
           W0_z, W1_z, b_z, W0_r, W1_r, b_r, W0_h, W1_h, b_h):
    pad = EPWP - EPW
    src = jnp.pad(edge_index[0].reshape(NW, EPW), ((0, 0), (0, pad)))
    dst = jnp.pad(edge_index[1].reshape(NW, EPW), ((0, 0), (0, pad)))
    w = jnp.pad(edge_weight.reshape(NW, EPW), ((0, 0), (0, pad)))
    src = src + (jnp.arange(NW, dtype=jnp.int32)[:, None] % NC) * N
    meta = jnp.stack(
        [src.reshape(NW, NCH, B), dst.reshape(NW, NCH, B)], axis=2)
    xx = jnp.concatenate([x, x], axis=0)
    agg_parts, deg_parts = _sc_agg(xx, meta, w.reshape(NW, NCH, B))
    return _tc_gru(
        x, agg_parts[0], agg_parts[1],
        deg_parts[0][:N, None], deg_parts[1][:N, None],
        W0_z[:F], W1_z[:F], W0_h[:F], W1_h[:F],
        b_z[None, :], b_h[None, :])


# speedup vs baseline: 35.7815x; 1.1442x over previous
import functools

import jax
import jax.numpy as jnp
from jax import lax
from jax.experimental import pallas as pl
from jax.experimental.pallas import tpu as pltpu
from jax.experimental.pallas import tpu_sc as plsc

N = 10000
F = 128
E = 320000

NC = 2
NS = 16
L = 16
NW = NC * NS
EPW = E // NW
B = 128
NCH = 80
EPWP = NCH * B
NP = 10240
NMETA = 4


def _sc_body(x_hbm, meta_hbm, w_hbm, agg_out, deg_out,
             r0, r1, m0, m1, m2, m3, w0, w1, w2, w3,
             acc_s, deg_s,
             g0, g1, s0, s1, d0, d1,
             gm0, gm1, gm2, gm3, gw0, gw1, gw2, gw3):
    cid = lax.axis_index("c")
    sid = lax.axis_index("s")
    wid = sid * NC + cid
    rows = [r0, r1]
    metab = [m0, m1, m2, m3]
    wbuf = [w0, w1, w2, w3]
    gs = [g0, g1]
    ss = [s0, s1]
    dsem = [d0, d1]
    gm = [gm0, gm1, gm2, gm3]
    gw = [gw0, gw1, gw2, gw3]

    zero16 = jnp.zeros((L,), jnp.float32)

    def _zrow(i, carry):
        for j in range(F // L):
            r0[i, pl.ds(j * L, L)] = zero16
        return carry
    lax.fori_loop(0, B, _zrow, 0)
    for t in range(NP // B // NS):
        zc = sid * (NP // B // NS) + t
        pltpu.sync_copy(r0, acc_s.at[pl.ds(zc * B, B)])
        pltpu.sync_copy(r0.at[0], deg_s.at[pl.ds(zc * B, B)])

    plsc.subcore_barrier()

    def _meta(c, i):
        pltpu.async_copy(meta_hbm.at[wid, c], metab[i], gm[i])
        pltpu.async_copy(w_hbm.at[wid, c], wbuf[i], gw[i])

    def _wait_meta(c, i):
        pltpu.make_async_copy(meta_hbm.at[wid, c], metab[i], gm[i]).wait()
        pltpu.make_async_copy(w_hbm.at[wid, c], wbuf[i], gw[i]).wait()

    def _gather(mi, buf, sem):
        return pltpu.async_copy(x_hbm.at[metab[mi].at[0]], buf, sem)

    def _wait_gather(mi, buf, sem):
        pltpu.make_async_copy(x_hbm.at[metab[mi].at[0]], buf, sem).wait()

    _meta(0, 0)
    _meta(1, 1)
    _wait_meta(0, 0)
    _gather(0, r0, g0)

    def _chunk_iter(k, carry):
        for i in range(NMETA):
            c = k * NMETA + i
            p = i % 2
            np_ = (i + 1) % 2
            mi = i
            mn = (i + 1) % NMETA
            m2_ = (i + 2) % NMETA

            _wait_gather(mi, rows[p], gs[p])

            @pl.when(c + 2 < NCH)
            def _():
                _meta(c + 2, m2_)

            @pl.when(c + 1 < NCH)
            def _():
                @pl.when(c >= 1)
                def _():
                    pltpu.make_async_copy(
                        rows[np_], acc_s.at[metab[mn].at[1]], ss[np_]).wait()
                    pltpu.make_async_copy(
                        wbuf[mn], deg_s.at[metab[mn].at[1]],
                        dsem[np_]).wait()
                _wait_meta(c + 1, mn)
                _gather(mn, rows[np_], gs[np_])

            def _grp(g, carry2):
                for k2 in range(L):
                    e = g * L + k2
                    wb = plsc.load_gather(
                        wbuf[mi], [jnp.full((L,), e, jnp.int32)])
                    for j in range(F // L):
                        rows[p][e, pl.ds(j * L, L)] = (
                            rows[p][e, pl.ds(j * L, L)] * wb)
                return carry2
            lax.fori_loop(0, B // L, _grp, 0)

            pltpu.async_copy(
                wbuf[mi], deg_s.at[metab[mi].at[1]], dsem[p], add=True)
            pltpu.async_copy(
                rows[p], acc_s.at[metab[mi].at[1]], ss[p], add=True)
        return carry
    lax.fori_loop(0, NCH // NMETA, _chunk_iter, 0)

    for p in range(2):
        pltpu.make_async_copy(
            rows[p], acc_s.at[metab[0].at[1]], ss[p]).wait()
        pltpu.make_async_copy(
            wbuf[0], deg_s.at[metab[0].at[1]], dsem[p]).wait()

    plsc.subcore_barrier()

    @pl.when(sid == 0)
    def _():
        pltpu.sync_copy(acc_s.at[pl.ds(0, N)], agg_out.at[cid])
        pltpu.sync_copy(deg_s, deg_out.at[cid])


_sc_agg = functools.partial(
    pl.kernel,
    out_type=(
        jax.ShapeDtypeStruct((NC, N, F), jnp.float32),
        jax.ShapeDtypeStruct((NC, NP), jnp.float32),
    ),
    mesh=plsc.VectorSubcoreMesh(core_axis_name="c", subcore_axis_name="s"),
    compiler_params=pltpu.CompilerParams(needs_layout_passes=False),
    scratch_types=(
        [pltpu.VMEM((B, F), jnp.float32)] * 2
        + [pltpu.VMEM((2, B), jnp.int32)] * NMETA
        + [pltpu.VMEM((B,), jnp.float32)] * NMETA
        + [
            pltpu.VMEM_SHARED((NP, F), jnp.float32),
            pltpu.VMEM_SHARED((NP,), jnp.float32),
        ]
        + [pltpu.SemaphoreType.DMA] * (6 + 2 * NMETA)
    ),
)(_sc_body)


RB = 1000


def _tc_body(x_ref, a0_ref, a1_ref, d0_ref, d1_ref,
             az_ref, bz_ref, ah_ref, bh_ref, vz_ref, vh_ref, o_ref):
    x = x_ref[...]
    agg = a0_ref[...] + a1_ref[...]
    deg = d0_ref[...] + d1_ref[...]
    deg_inv = jnp.where(deg > 0, 1.0 / deg, 0.0)
    agg = agg * deg_inv
    pz = (jnp.dot(x, az_ref[...], preferred_element_type=jnp.float32)
          + jnp.dot(agg, bz_ref[...], preferred_element_type=jnp.float32)
          + vz_ref[...])
    ph = (jnp.dot(x, ah_ref[...], preferred_element_type=jnp.float32)
          + jnp.dot(agg, bh_ref[...], preferred_element_type=jnp.float32)
          + vh_ref[...])
    z = jax.nn.sigmoid(pz)
    ht = jnp.tanh(ph)
    o_ref[...] = jnp.maximum((1.0 - z) * ht, 0.0)


def _tc_gru(x, a0, a1, d0, d1, az, bz, ah, bh, vz, vh):
    grid = (N // RB,)
    row = pl.BlockSpec((RB, F), lambda i: (i, 0))
    col = pl.BlockSpec((RB, 1), lambda i: (i, 0))
    full = pl.BlockSpec((F, F), lambda i: (0, 0))
    vec = pl.BlockSpec((1, F), lambda i: (0, 0))
    return pl.pallas_call(
        _tc_body,
        grid=grid,
        in_specs=[row, row, row, col, col, full, full, full, full, vec, vec],
        out_specs=row,
        out_shape=jax.ShapeDtypeStruct((N, F), jnp.float32),
    )(x, a0, a1, d0, d1, az, bz, ah, bh, vz, vh)


def kernel(x, edge_index, edge_weight,
           W0_z, W1_z, b_z, W0_r, W1_r, b_r, W0_h, W1_h, b_h):
    pad = EPWP - EPW
    src = jnp.pad(edge_index[0].reshape(NW, EPW), ((0, 0), (0, pad)))
    dst = jnp.pad(edge_index[1].reshape(NW, EPW), ((0, 0), (0, pad)))
    w = jnp.pad(edge_weight.reshape(NW, EPW), ((0, 0), (0, pad)))
    wids = jnp.arange(NW, dtype=jnp.int32)
    copy_id = (wids % NC) * 2 + (wids // NC) % 2
    src = src + copy_id[:, None] * N
    meta = jnp.stack(
        [src.reshape(NW, NCH, B), dst.reshape(NW, NCH, B)], axis=2)
    xx = jnp.concatenate([x, x, x, x], axis=0)
    agg_parts, deg_parts = _sc_agg(xx, meta, w.reshape(NW, NCH, B))
    return _tc_gru(
        x, agg_parts[0], agg_parts[1],
        deg_parts[0][:N, None], deg_parts[1][:N, None],
        W0_z[:F], W1_z[:F], W0_h[:F], W1_h[:F],
        b_z[None, :], b_h[None, :])


# speedup vs baseline: 42.8987x; 1.1989x over previous
import functools

import jax
import jax.numpy as jnp
from jax import lax
from jax.experimental import pallas as pl
from jax.experimental.pallas import tpu as pltpu
from jax.experimental.pallas import tpu_sc as plsc

N = 10000
F = 128
E = 320000

NC = 2
NS = 16
L = 16
NW = NC * NS
EPW = E // NW
B = 128
NCH = 80
EPWP = NCH * B
NP = 10240
NMETA = 4


def _sc_body(x_hbm, meta_hbm, w_hbm, agg_out, deg_out,
             r0, r1, m0, m1, m2, m3, w0, w1, w2, w3,
             acc_s, deg_s,
             g0, g1, s0, s1, d0, d1,
             gm0, gm1, gm2, gm3, gw0, gw1, gw2, gw3):
    cid = lax.axis_index("c")
    sid = lax.axis_index("s")
    wid = sid * NC + cid
    rows = [r0, r1]
    metab = [m0, m1, m2, m3]
    wbuf = [w0, w1, w2, w3]
    gs = [g0, g1]
    ss = [s0, s1]
    dsem = [d0, d1]
    gm = [gm0, gm1, gm2, gm3]
    gw = [gw0, gw1, gw2, gw3]

    zero16 = jnp.zeros((L,), jnp.float32)

    def _zrow(i, carry):
        for j in range(F // L):
            r0[i, pl.ds(j * L, L)] = zero16
        return carry
    lax.fori_loop(0, B, _zrow, 0)
    for t in range(NP // B // NS):
        zc = sid * (NP // B // NS) + t
        pltpu.sync_copy(r0, acc_s.at[pl.ds(zc * B, B)])
        pltpu.sync_copy(r0.at[0], deg_s.at[pl.ds(zc * B, B)])

    plsc.subcore_barrier()

    def _meta(c, i):
        pltpu.async_copy(meta_hbm.at[wid, c], metab[i], gm[i])
        pltpu.async_copy(w_hbm.at[wid, c], wbuf[i], gw[i])

    def _wait_meta(c, i):
        pltpu.make_async_copy(meta_hbm.at[wid, c], metab[i], gm[i]).wait()
        pltpu.make_async_copy(w_hbm.at[wid, c], wbuf[i], gw[i]).wait()

    def _gather(mi, buf, sem):
        return pltpu.async_copy(x_hbm.at[metab[mi].at[0]], buf, sem)

    def _wait_gather(mi, buf, sem):
        pltpu.make_async_copy(x_hbm.at[metab[mi].at[0]], buf, sem).wait()

    _meta(0, 0)
    _meta(1, 1)
    _wait_meta(0, 0)
    _gather(0, r0, g0)

    def _chunk_iter(k, carry):
        for i in range(NMETA):
            c = k * NMETA + i
            p = i % 2
            np_ = (i + 1) % 2
            mi = i
            mn = (i + 1) % NMETA
            m2_ = (i + 2) % NMETA

            _wait_gather(mi, rows[p], gs[p])

            @pl.when(c + 2 < NCH)
            def _():
                _meta(c + 2, m2_)

            @pl.when(c + 1 < NCH)
            def _():
                @pl.when(c >= 1)
                def _():
                    pltpu.make_async_copy(
                        rows[np_], acc_s.at[metab[mn].at[1]], ss[np_]).wait()
                    pltpu.make_async_copy(
                        wbuf[mn], deg_s.at[metab[mn].at[1]],
                        dsem[np_]).wait()
                _wait_meta(c + 1, mn)
                _gather(mn, rows[np_], gs[np_])

            def _grp(g, carry2):
                for k2 in range(L):
                    e = g * L + k2
                    wb = plsc.load_gather(
                        wbuf[mi], [jnp.full((L,), e, jnp.int32)])
                    for j in range(F // L):
                        rows[p][e, pl.ds(j * L, L)] = (
                            rows[p][e, pl.ds(j * L, L)] * wb)
                return carry2
            lax.fori_loop(0, B // L, _grp, 0)

            pltpu.async_copy(
                wbuf[mi], deg_s.at[metab[mi].at[1]], dsem[p], add=True)
            pltpu.async_copy(
                rows[p], acc_s.at[metab[mi].at[1]], ss[p], add=True)
        return carry
    lax.fori_loop(0, NCH // NMETA, _chunk_iter, 0)

    for p in range(2):
        pltpu.make_async_copy(
            rows[p], acc_s.at[metab[0].at[1]], ss[p]).wait()
        pltpu.make_async_copy(
            wbuf[0], deg_s.at[metab[0].at[1]], dsem[p]).wait()

    plsc.subcore_barrier()

    @pl.when(sid == 0)
    def _():
        pltpu.sync_copy(acc_s.at[pl.ds(0, N)], agg_out.at[cid])
        pltpu.sync_copy(deg_s, deg_out.at[cid])


_sc_agg = functools.partial(
    pl.kernel,
    out_type=(
        jax.ShapeDtypeStruct((NC, N, F), jnp.float32),
        jax.ShapeDtypeStruct((NC, NP), jnp.float32),
    ),
    mesh=plsc.VectorSubcoreMesh(core_axis_name="c", subcore_axis_name="s"),
    compiler_params=pltpu.CompilerParams(needs_layout_passes=False),
    scratch_types=(
        [pltpu.VMEM((B, F), jnp.float32)] * 2
        + [pltpu.VMEM((2, B), jnp.int32)] * NMETA
        + [pltpu.VMEM((B,), jnp.float32)] * NMETA
        + [
            pltpu.VMEM_SHARED((NP, F), jnp.float32),
            pltpu.VMEM_SHARED((NP,), jnp.float32),
        ]
        + [pltpu.SemaphoreType.DMA] * (6 + 2 * NMETA)
    ),
)(_sc_body)


RB = 1000


def _tc_body(x_ref, a0_ref, a1_ref, d0_ref, d1_ref,
             az_ref, bz_ref, ah_ref, bh_ref, vz_ref, vh_ref, o_ref):
    x = x_ref[...]
    agg = a0_ref[...] + a1_ref[...]
    deg = d0_ref[...] + d1_ref[...]
    deg_inv = jnp.where(deg > 0, 1.0 / deg, 0.0)
    agg = agg * deg_inv
    pz = (jnp.dot(x, az_ref[...], preferred_element_type=jnp.float32)
          + jnp.dot(agg, bz_ref[...], preferred_element_type=jnp.float32)
          + vz_ref[...])
    ph = (jnp.dot(x, ah_ref[...], preferred_element_type=jnp.float32)
          + jnp.dot(agg, bh_ref[...], preferred_element_type=jnp.float32)
          + vh_ref[...])
    z = jax.nn.sigmoid(pz)
    ht = jnp.tanh(ph)
    o_ref[...] = jnp.maximum((1.0 - z) * ht, 0.0)


def _tc_gru(x, a0, a1, d0, d1, az, bz, ah, bh, vz, vh):
    grid = (N // RB,)
    row = pl.BlockSpec((RB, F), lambda i: (i, 0))
    col = pl.BlockSpec((RB, 1), lambda i: (i, 0))
    full = pl.BlockSpec((F, F), lambda i: (0, 0))
    vec = pl.BlockSpec((1, F), lambda i: (0, 0))
    return pl.pallas_call(
        _tc_body,
        grid=grid,
        in_specs=[row, row, row, col, col, full, full, full, full, vec, vec],
        out_specs=row,
        out_shape=jax.ShapeDtypeStruct((N, F), jnp.float32),
    )(x, a0, a1, d0, d1, az, bz, ah, bh, vz, vh)


def kernel(x, edge_index, edge_weight,
           W0_z, W1_z, b_z, W0_r, W1_r, b_r, W0_h, W1_h, b_h):
    pad = EPWP - EPW
    src = jnp.pad(edge_index[0].reshape(NW, EPW), ((0, 0), (0, pad)))
    dst = jnp.pad(edge_index[1].reshape(NW, EPW), ((0, 0), (0, pad)))
    w = jnp.pad(edge_weight.reshape(NW, EPW), ((0, 0), (0, pad)))
    wids = jnp.arange(NW, dtype=jnp.int32)
    copy_id = (wids % NC) * 4 + (wids // NC) % 4
    src = src + copy_id[:, None] * N
    meta = jnp.stack(
        [src.reshape(NW, NCH, B), dst.reshape(NW, NCH, B)], axis=2)
    xx = jnp.concatenate([x] * 8, axis=0)
    agg_parts, deg_parts = _sc_agg(xx, meta, w.reshape(NW, NCH, B))
    return _tc_gru(
        x, agg_parts[0], agg_parts[1],
        deg_parts[0][:N, None], deg_parts[1][:N, None],
        W0_z[:F], W1_z[:F], W0_h[:F], W1_h[:F],
        b_z[None, :], b_h[None, :])
